# Initial kernel scaffold; baseline (speedup 1.0000x reference)
import jax
import jax.numpy as jnp
from jax.experimental import pallas as pl


def kernel(x, cu_seqlens_q, cu_seqlens_k, k_cache, v_cache, cmp_k_cache, cmp_v_cache, Wq, Wk, Wv, Wo, Wck, Wcv, pe, Wg):
    raise NotImplementedError("write your pallas kernel here")


# speedup vs baseline: 1.2497x; 1.2497x over previous
import functools
import numpy as np
import jax
import jax.numpy as jnp
from jax.experimental import pallas as pl
from jax.experimental.pallas import tpu as pltpu

B = 32; S = 2048; S_CACHE = S - 1; H = 4096; HQ = 32; HKV = 4; D = 128
KS = 32; KST = 16; BS = 64; TOPK = 16; WIN = 512
THETA = 10000.0
G = HQ // HKV
CMP = (S - KS) // KST + 1
CMP_CACHE = (S_CACHE - KS) // KST + 1
NBLK = S // BS
SCALE = 1.0 / float(np.sqrt(D))
HALF = D // 2
NEG = -1e30
HIGH = jax.lax.Precision.DEFAULT

_INV = (np.float32(1.0)
        / (np.float32(THETA) ** (np.arange(HALF, dtype=np.float32)
                                 / np.float32(HALF)))).astype(np.float32)
_ANG = np.arange(S, dtype=np.float32)[:, None] * _INV[None, :]
_TAB = np.concatenate([np.cos(_ANG).astype(np.float32),
                       np.sin(_ANG).astype(np.float32)], axis=1)
_CMP_ANG = (np.arange(CMP, dtype=np.float32) * np.float32(KST))[:, None] * _INV[None, :]
_CMP_TAB = np.concatenate([np.cos(_CMP_ANG).astype(np.float32),
                           np.sin(_CMP_ANG).astype(np.float32)], axis=1)

WBLK0 = (S - 1 - WIN) // BS
NWEXT = NBLK - 2 - WBLK0
NSLOT = TOPK + NWEXT
NTOK = NSLOT * BS


def _rope_tab(x, tab):
    c = tab[..., :HALF]
    sn = tab[..., HALF:]
    x1 = x[..., :HALF]
    x2 = x[..., HALF:]
    return jnp.concatenate([x1 * c - x2 * sn, x2 * c + x1 * sn], axis=-1)


def _qproj_body(x_ref, w_ref, t_ref, o_ref):
    y = jnp.dot(x_ref[...], w_ref[...], preferred_element_type=jnp.float32,
                precision=HIGH)
    y = y.reshape(B, -1, D)
    o_ref[...] = _rope_tab(y, t_ref[...][None]).reshape(B, -1)


def _qproj(xf, Wq):
    NJ = 8
    CB = (HQ * D) // NJ
    return pl.pallas_call(
        _qproj_body,
        grid=(NJ,),
        in_specs=[pl.BlockSpec((B, H), lambda j: (0, 0)),
                  pl.BlockSpec((H, CB), lambda j: (0, j)),
                  pl.BlockSpec((1, D), lambda j: (0, 0))],
        out_specs=pl.BlockSpec((B, CB), lambda j: (0, j)),
        out_shape=jax.ShapeDtypeStruct((B, HQ * D), jnp.float32),
    )(xf, Wq, jnp.asarray(_TAB[S - 1:S]))


def _kvg_body(x_ref, wk_ref, wv_ref, wg_ref, kt_ref, vt_ref, v4_ref, g_ref):
    x = x_ref[...]
    for h in range(HKV):
        kt_ref[h] = jnp.dot(x, wk_ref[:, h * D:(h + 1) * D],
                            preferred_element_type=jnp.float32, precision=HIGH)
        vh = jnp.dot(x, wv_ref[:, h * D:(h + 1) * D],
                     preferred_element_type=jnp.float32, precision=HIGH)
        vt_ref[h] = vh
        v4_ref[:, h, 0, :] = vh
    g_ref[...] = jax.nn.sigmoid(jnp.dot(x, wg_ref[...],
                                        preferred_element_type=jnp.float32,
                                        precision=HIGH))


def _kvg(xf, Wk, Wv, Wg):
    return pl.pallas_call(
        _kvg_body,
        in_specs=[pl.BlockSpec((B, H), lambda: (0, 0)),
                  pl.BlockSpec((H, HKV * D), lambda: (0, 0)),
                  pl.BlockSpec((H, HKV * D), lambda: (0, 0)),
                  pl.BlockSpec((H, 3), lambda: (0, 0))],
        out_specs=[pl.BlockSpec((HKV, B, D), lambda: (0, 0, 0)),
                   pl.BlockSpec((HKV, B, D), lambda: (0, 0, 0)),
                   pl.BlockSpec((B, HKV, 1, D), lambda: (0, 0, 0, 0)),
                   pl.BlockSpec((B, 3), lambda: (0, 0))],
        out_shape=[jax.ShapeDtypeStruct((HKV, B, D), jnp.float32),
                   jax.ShapeDtypeStruct((HKV, B, D), jnp.float32),
                   jax.ShapeDtypeStruct((B, HKV, 1, D), jnp.float32),
                   jax.ShapeDtypeStruct((B, 3), jnp.float32)],
    )(xf, Wk, Wv, Wg)


def _cmp_body(wkc_ref, wvc_ref, knew_ref, vnew_ref, pe_ref, wck_ref, wcv_ref,
              ckc_ref, cvc_ref, q_ref, ct_ref, lt_ref, cout_ref, topk_ref, knr_ref):
    k_new = knew_ref[0]
    win_k = jnp.concatenate(
        [wkc_ref[...].astype(jnp.float32).reshape(B, KS - 1, D),
         k_new[:, None, :]], axis=1)
    win_v = jnp.concatenate(
        [wvc_ref[...].astype(jnp.float32).reshape(B, KS - 1, D),
         vnew_ref[0][:, None, :]], axis=1)
    win_k = win_k + pe_ref[0][None, :, :]
    wkf = win_k.reshape(B, KS * D)
    wvf = win_v.reshape(B, KS * D)
    ck_new = jnp.dot(wkf, wck_ref[0], preferred_element_type=jnp.float32,
                     precision=HIGH)
    cv_new = jnp.dot(wvf, wcv_ref[0], preferred_element_type=jnp.float32)
    cmp_k = jnp.concatenate(
        [ckc_ref[...].astype(jnp.float32).reshape(B, CMP_CACHE, D),
         ck_new[:, None, :]], axis=1)
    cmp_v = jnp.concatenate(
        [cvc_ref[...].astype(jnp.float32).reshape(B, CMP_CACHE, D),
         cv_new[:, None, :]], axis=1)
    cmp_kr = _rope_tab(cmp_k, ct_ref[...][None])
    q = q_ref[:, 0, :, :]
    cs = jax.lax.dot_general(q, cmp_kr, (((2,), (2,)), ((0,), (0,))),
                             precision=HIGH) * SCALE
    m = jnp.max(cs, axis=-1, keepdims=True)
    p = jnp.exp(cs - m)
    cp = p / jnp.sum(p, axis=-1, keepdims=True)
    cout_ref[:, 0, :, :] = jax.lax.dot_general(cp, cmp_v, (((2,), (1,)), ((0,), (0,))))
    cpt = cp[:, 0]
    for g in range(1, G):
        cpt = cpt + cp[:, g]
    r4 = cpt[:, : (NBLK - 1) * 4].reshape(B, NBLK - 1, 4)
    b_main = ((r4[:, :, 0] + r4[:, :, 1]) + r4[:, :, 2]) + r4[:, :, 3]
    b_last = ((cpt[:, (NBLK - 1) * 4] + cpt[:, (NBLK - 1) * 4 + 1])
              + cpt[:, (NBLK - 1) * 4 + 2])[:, None]
    bs = jnp.concatenate([b_main, b_last], axis=1)
    col = jax.lax.broadcasted_iota(jnp.int32, (1, NBLK), 1)
    forced = (col == 0) | (col >= NBLK - 2)
    bs = jnp.where(forced, jnp.inf, bs)
    gt = (bs[:, None, :] > bs[:, :, None]).astype(jnp.float32)
    eq = ((bs[:, None, :] == bs[:, :, None])
          & (col[0][None, None, :] < col[0][None, :, None])).astype(jnp.float32)
    rank = jnp.sum(gt + eq, axis=2)
    sel = rank < TOPK
    tri = (jax.lax.broadcasted_iota(jnp.int32, (NBLK, NBLK), 0)
           <= jax.lax.broadcasted_iota(jnp.int32, (NBLK, NBLK), 1)).astype(jnp.float32)
    pos = jnp.dot(sel.astype(jnp.float32), tri, preferred_element_type=jnp.float32) - 1.0
    tcol = jax.lax.broadcasted_iota(jnp.int32, (1, 1, TOPK), 2).astype(jnp.float32)
    onehot = sel[:, :, None] & (pos[:, :, None] == tcol)
    ivec = jax.lax.broadcasted_iota(jnp.int32, (1, NBLK, 1), 1).astype(jnp.float32)
    topk_ref[0] = jnp.sum(jnp.where(onehot, ivec, 0.0), axis=1).astype(jnp.int32)
    knr_ref[:, 0, 0, :] = _rope_tab(k_new, lt_ref[...])


def _cmp_topk(win_kc, win_vc, k_t, v_t, pe, Wck, Wcv, ckc, cvc, qr4):
    NW = B * (KS - 1)
    NC = B * CMP_CACHE
    return pl.pallas_call(
        _cmp_body,
        grid=(HKV,),
        in_specs=[
            pl.BlockSpec((NW, D), lambda h: (0, h)),
            pl.BlockSpec((NW, D), lambda h: (0, h)),
            pl.BlockSpec((1, B, D), lambda h: (h, 0, 0)),
            pl.BlockSpec((1, B, D), lambda h: (h, 0, 0)),
            pl.BlockSpec((1, KS, D), lambda h: (h, 0, 0)),
            pl.BlockSpec((1, KS * D, D), lambda h: (h, 0, 0)),
            pl.BlockSpec((1, KS * D, D), lambda h: (h, 0, 0)),
            pl.BlockSpec((NC, D), lambda h: (0, h)),
            pl.BlockSpec((NC, D), lambda h: (0, h)),
            pl.BlockSpec((B, 1, G, D), lambda h: (0, h, 0, 0)),
            pl.BlockSpec((CMP, D), lambda h: (0, 0)),
            pl.BlockSpec((1, D), lambda h: (0, 0)),
        ],
        out_specs=[
            pl.BlockSpec((B, 1, G, D), lambda h: (0, h, 0, 0)),
            pl.BlockSpec((1, B, TOPK), lambda h: (h, 0, 0)),
            pl.BlockSpec((B, 1, 1, D), lambda h: (0, h, 0, 0)),
        ],
        out_shape=[
            jax.ShapeDtypeStruct((B, HKV, G, D), jnp.float32),
            jax.ShapeDtypeStruct((HKV, B, TOPK), jnp.int32),
            jax.ShapeDtypeStruct((B, HKV, 1, D), jnp.float32),
        ],
    )(win_kc, win_vc, k_t, v_t, pe, Wck, Wcv, ckc, cvc, qr4,
      jnp.asarray(_CMP_TAB), jnp.asarray(_TAB[S - 1:S]))


def _mk_slot_idx(j):
    if j < TOPK:
        def f(bh, tp):
            return (bh // HKV, tp[bh * TOPK + j], bh % HKV)
    else:
        def f(bh, tp):
            return (bh // HKV, WBLK0 + (j - TOPK), bh % HKV)
    return f


def _attn_body(tp_ref, *refs):
    k_refs = refs[:NSLOT]
    v_refs = refs[NSLOT:2 * NSLOT]
    tab_ref, q_ref, knr_ref, vnew_ref, gate_ref, cout_ref, o_ref = refs[2 * NSLOT:]
    bh = pl.program_id(0)

    krs = []
    vrs = []
    toks = []
    for j in range(NSLOT):
        if j < TOPK:
            blk = tp_ref[bh * TOPK + j]
        else:
            blk = WBLK0 + (j - TOPK)
        kf = k_refs[j][0].astype(jnp.float32)
        tab = tab_ref[pl.ds(blk * BS, BS), :]
        c = tab[:, :HALF]
        sn = tab[:, HALF:]
        k1 = kf[:, :HALF]
        k2 = kf[:, HALF:]
        krs.append(jnp.concatenate([k1 * c - k2 * sn, k2 * c + k1 * sn], axis=1))
        vf = v_refs[j][0].astype(jnp.float32)
        tok_c = blk * BS + jax.lax.broadcasted_iota(jnp.int32, (BS, 1), 0)
        vrs.append(jnp.where(tok_c == (S - 1), 0.0, vf))
        toks.append(blk * BS + jax.lax.broadcasted_iota(jnp.int32, (1, BS), 1))
    kr = jnp.concatenate(krs, axis=0)
    vr = jnp.concatenate(vrs, axis=0)
    tok = jnp.concatenate(toks, axis=1)
    slot = jax.lax.broadcasted_iota(jnp.int32, (1, NTOK), 1) // BS
    valid = tok != (S - 1)

    q = q_ref[0, 0]
    sc = jax.lax.dot_general(q, kr, (((1,), (1,)), ((), ()))) * SCALE

    knr = knr_ref[0, 0, 0]
    vnew = vnew_ref[0, 0, 0]
    sn_sc = jnp.sum(q * knr[None, :], axis=1, keepdims=True) * SCALE

    def _masked_attn(mask):
        scm = jnp.where(mask, sc, NEG)
        m = jnp.max(scm, axis=1, keepdims=True)
        m2 = jnp.maximum(m, sn_sc)
        p = jnp.where(mask, jnp.exp(sc - m2), 0.0)
        l = jnp.sum(p, axis=1, keepdims=True)
        o = jnp.dot(p, vr, preferred_element_type=jnp.float32)
        bnew = jnp.exp(sn_sc - m2)
        return (o + bnew * vnew[None, :]) / (l + bnew)

    sp_mask = (slot < TOPK) & valid
    wn_mask = (slot >= TOPK - 2) & (tok >= S - 1 - WIN) & valid
    sp_out = _masked_attn(sp_mask)
    wn_out = _masked_attn(wn_mask)
    g0 = gate_ref[0, 0, 0]
    g1 = gate_ref[0, 0, 1]
    g2 = gate_ref[0, 0, 2]
    o_ref[0, 0] = g0 * cout_ref[0, 0] + g1 * sp_out + g2 * wn_out


def _sparse_win_attn(tp, kf2, vf2, tab, qr4, knr, v_new4, gate3, cout):
    blk_specs = [pl.BlockSpec((1, BS, D), _mk_slot_idx(j)) for j in range(NSLOT)]
    return pl.pallas_call(
        _attn_body,
        grid_spec=pltpu.PrefetchScalarGridSpec(
            num_scalar_prefetch=1,
            grid=(B * HKV,),
            in_specs=blk_specs + blk_specs + [
                pl.BlockSpec((S, D), lambda bh, tp: (0, 0)),
                pl.BlockSpec((1, 1, G, D), lambda bh, tp: (bh // HKV, bh % HKV, 0, 0)),
                pl.BlockSpec((1, 1, 1, D), lambda bh, tp: (bh // HKV, bh % HKV, 0, 0)),
                pl.BlockSpec((1, 1, 1, D), lambda bh, tp: (bh // HKV, bh % HKV, 0, 0)),
                pl.BlockSpec((1, 1, 3), lambda bh, tp: (bh // HKV, 0, 0)),
                pl.BlockSpec((1, 1, G, D), lambda bh, tp: (bh // HKV, bh % HKV, 0, 0)),
            ],
            out_specs=pl.BlockSpec((1, 1, G, D),
                                   lambda bh, tp: (bh // HKV, bh % HKV, 0, 0)),
        ),
        out_shape=jax.ShapeDtypeStruct((B, HKV, G, D), jnp.float32),
        compiler_params=pltpu.CompilerParams(dimension_semantics=("arbitrary",)),
    )(tp, *([kf2] * NSLOT), *([vf2] * NSLOT), tab, qr4, knr, v_new4, gate3, cout)


def _oproj_body(x_ref, w_ref, o_ref):
    o_ref[...] = jnp.dot(x_ref[...], w_ref[...], preferred_element_type=jnp.float32)


def _oproj(o, Wo):
    NJ = 8
    CB = H // NJ
    return pl.pallas_call(
        _oproj_body,
        grid=(NJ,),
        in_specs=[pl.BlockSpec((B, HQ * D), lambda j: (0, 0)),
                  pl.BlockSpec((HQ * D, CB), lambda j: (0, j))],
        out_specs=pl.BlockSpec((B, CB), lambda j: (0, j)),
        out_shape=jax.ShapeDtypeStruct((B, H), jnp.float32),
    )(o, Wo)


def kernel(x, cu_seqlens_q, cu_seqlens_k, k_cache, v_cache, cmp_k_cache, cmp_v_cache,
           Wq, Wk, Wv, Wo, Wck, Wcv, pe, Wg):
    xf = x.astype(jnp.float32)
    qr = _qproj(xf, Wq)
    k_t, v_t, v_new4, gate = _kvg(xf, Wk, Wv, Wg)

    kf2 = k_cache.reshape(B, S_CACHE, HKV * D)
    vf2 = v_cache.reshape(B, S_CACHE, HKV * D)
    start = CMP_CACHE * KST
    win_kc = kf2[:, start:, :].reshape(B * (KS - 1), HKV * D)
    win_vc = vf2[:, start:, :].reshape(B * (KS - 1), HKV * D)
    ckc = cmp_k_cache.reshape(B * CMP_CACHE, HKV * D)
    cvc = cmp_v_cache.reshape(B * CMP_CACHE, HKV * D)
    wck = Wck.reshape(HKV, KS * D, D)
    wcv = Wcv.reshape(HKV, KS * D, D)
    qr4 = qr.reshape(B, HKV, G, D)

    cout, topk, knr = _cmp_topk(win_kc, win_vc, k_t, v_t, pe, wck, wcv,
                                ckc, cvc, qr4)
    tp = topk.transpose(1, 0, 2).reshape(-1)
    tab = jnp.asarray(_TAB)
    gate3 = gate.reshape(B, 1, 3)
    o = _sparse_win_attn(tp, kf2, vf2, tab, qr4, knr, v_new4, gate3, cout)
    return _oproj(o.reshape(B, HQ * D), Wo)
